# Initial kernel scaffold; baseline (speedup 1.0000x reference)
import jax
import jax.numpy as jnp
from jax.experimental import pallas as pl


def kernel(x, pos, edge_index, edge_attr, W1, b1, W2, b2, Wn1, bn1, Wn2, bn2, Wc1, bc1, Wc2, bc2):
    raise NotImplementedError("write your pallas kernel here")


# speedup vs baseline: 2.8658x; 2.8658x over previous
import functools

import jax
import jax.numpy as jnp
from jax import lax
from jax.experimental import pallas as pl
from jax.experimental.pallas import tpu as pltpu
from jax.experimental.pallas import tpu_sc as plsc

NC = 2
NS = 16
NW = NC * NS
GW = 80
CHUNK = 80


def _silu(v):
    return v * jax.nn.sigmoid(v)


def _pre_body(x_ref, posp_ref, w1a_ref, b1_ref, w1b_ref, wn1a_ref, bn1_ref,
              xap_ref, xbp_ref, xn_ref):
    x = x_ref[...]
    posp = posp_ref[...]
    xa = jnp.dot(x, w1a_ref[...], preferred_element_type=jnp.float32) + b1_ref[...]
    xb = jnp.dot(x, w1b_ref[...], preferred_element_type=jnp.float32)
    xap_ref[...] = jnp.concatenate([xa, posp], axis=1)
    xbp_ref[...] = jnp.concatenate([xb, posp], axis=1)
    xn_ref[...] = jnp.dot(x, wn1a_ref[...], preferred_element_type=jnp.float32) + bn1_ref[...]


def _edge_body(ga_ref, gb_ref, ea_ref, w1c_ref, w1d_ref, w2_ref, b2_ref,
               wc1_ref, bc1_ref, wc2_ref, bc2_ref, md_ref):
    ga = ga_ref[...]
    gb = gb_ref[...]
    diffp = ga[:, 64:80] - gb[:, 64:80]
    dist = jnp.sqrt(jnp.sum(diffp * diffp, axis=1, keepdims=True))
    pre = (ga[:, :64] + gb[:, :64]
           + dist * w1c_ref[...]
           + jnp.dot(ea_ref[...], w1d_ref[...], preferred_element_type=jnp.float32))
    msg = _silu(jnp.dot(_silu(pre), w2_ref[...], preferred_element_type=jnp.float32)
                + b2_ref[...])
    c1 = _silu(jnp.dot(msg, wc1_ref[...], preferred_element_type=jnp.float32)
               + bc1_ref[...])
    cw = jnp.dot(c1, wc2_ref[...], preferred_element_type=jnp.float32) + bc2_ref[...]
    md_ref[...] = jnp.concatenate([msg, diffp * cw], axis=1)


def _node_body(acc0_ref, acc1_ref, xn_ref, posp_ref, wn1b_ref, wn2_ref, bn2_ref,
               xnew_ref, posn_ref):
    acc = acc0_ref[...] + acc1_ref[...]
    h = _silu(xn_ref[...] + jnp.dot(acc[:, :64], wn1b_ref[...],
                                    preferred_element_type=jnp.float32))
    xnew_ref[...] = jnp.dot(h, wn2_ref[...], preferred_element_type=jnp.float32) + bn2_ref[...]
    posn_ref[...] = posp_ref[...] + acc[:, 64:80]


def _make_gather(E, N):
    epw = E // NW
    nch = epw // CHUNK
    mesh = plsc.VectorSubcoreMesh(core_axis_name="c", subcore_axis_name="s",
                                  num_cores=NC, num_subcores=NS)

    @functools.partial(
        pl.kernel, mesh=mesh,
        compiler_params=pltpu.CompilerParams(use_tc_tiling_on_sc=False),
        out_type=[jax.ShapeDtypeStruct((E, GW), jnp.float32),
                  jax.ShapeDtypeStruct((E, GW), jnp.float32)],
        scratch_types=[pltpu.VMEM((CHUNK,), jnp.int32),
                       pltpu.VMEM((CHUNK,), jnp.int32),
                       pltpu.VMEM((CHUNK, GW), jnp.float32),
                       pltpu.VMEM((CHUNK, GW), jnp.float32),
                       pltpu.SemaphoreType.DMA,
                       pltpu.SemaphoreType.DMA],
    )
    def gather_k(xap_hbm, xbp_hbm, row_hbm, col_hbm, ga_hbm, gb_hbm,
                 idx_r, idx_c, buf_a, buf_b, sem_a, sem_b):
        wid = lax.axis_index("s") * NC + lax.axis_index("c")
        base = wid * epw

        def body(i, carry):
            off = base + i * CHUNK
            pltpu.sync_copy(row_hbm.at[pl.ds(off, CHUNK)], idx_r)
            pltpu.sync_copy(col_hbm.at[pl.ds(off, CHUNK)], idx_c)
            cp_a = pltpu.async_copy(xap_hbm.at[idx_r], buf_a, sem_a)
            cp_b = pltpu.async_copy(xbp_hbm.at[idx_c], buf_b, sem_b)
            cp_a.wait()
            cp_b.wait()
            pltpu.sync_copy(buf_a, ga_hbm.at[pl.ds(off, CHUNK)])
            pltpu.sync_copy(buf_b, gb_hbm.at[pl.ds(off, CHUNK)])
            return carry

        lax.fori_loop(0, nch, body, 0)

    return gather_k


def _make_scatter(E, N):
    epw = E // NW
    nch = epw // CHUNK
    npc = N // NS
    mesh = plsc.VectorSubcoreMesh(core_axis_name="c", subcore_axis_name="s",
                                  num_cores=NC, num_subcores=NS)

    @functools.partial(
        pl.kernel, mesh=mesh,
        compiler_params=pltpu.CompilerParams(use_tc_tiling_on_sc=False),
        out_type=jax.ShapeDtypeStruct((NC, N, GW), jnp.float32),
        scratch_types=[pltpu.VMEM((CHUNK,), jnp.int32),
                       pltpu.VMEM((CHUNK, GW), jnp.float32),
                       pltpu.VMEM_SHARED((N, GW), jnp.float32)],
    )
    def scatter_k(md_hbm, row_hbm, zeros_hbm, acc_hbm, idx_v, buf, acc_sh):
        cid = lax.axis_index("c")
        sid = lax.axis_index("s")
        wid = sid * NC + cid
        pltpu.sync_copy(zeros_hbm.at[pl.ds(sid * npc, npc)],
                        acc_sh.at[pl.ds(sid * npc, npc)])
        plsc.subcore_barrier()
        base = wid * epw

        def body(i, carry):
            off = base + i * CHUNK
            pltpu.sync_copy(row_hbm.at[pl.ds(off, CHUNK)], idx_v)
            pltpu.sync_copy(md_hbm.at[pl.ds(off, CHUNK)], buf)
            pltpu.sync_copy(buf, acc_sh.at[idx_v], add=True)
            return carry

        lax.fori_loop(0, nch, body, 0)
        plsc.subcore_barrier()
        pltpu.sync_copy(acc_sh.at[pl.ds(sid * npc, npc)],
                        acc_hbm.at[cid, pl.ds(sid * npc, npc)])

    return scatter_k


def kernel(x, pos, edge_index, edge_attr, W1, b1, W2, b2,
           Wn1, bn1, Wn2, bn2, Wc1, bc1, Wc2, bc2):
    N, D = x.shape
    E = edge_index.shape[1]
    H = W2.shape[0]
    assert D == 128 and H == 64
    assert E % (NW * CHUNK) == 0 and N % NS == 0

    row = edge_index[0]
    col = edge_index[1]
    posp = jnp.pad(pos, ((0, 0), (0, 16 - pos.shape[1])))
    w1a = W1[:D]
    w1b = W1[D:2 * D]
    w1c = W1[2 * D:2 * D + 1]
    w1d = W1[2 * D + 1:]
    wn1a = Wn1[:D]
    wn1b = Wn1[D:]
    wc2t = jnp.tile(Wc2, (1, 16))
    bc2t = jnp.broadcast_to(bc2.reshape(1, 1), (1, 16))

    bpre = 2000
    xap, xbp, xn = pl.pallas_call(
        _pre_body,
        grid=(N // bpre,),
        in_specs=[
            pl.BlockSpec((bpre, D), lambda i: (i, 0)),
            pl.BlockSpec((bpre, 16), lambda i: (i, 0)),
            pl.BlockSpec((D, H), lambda i: (0, 0)),
            pl.BlockSpec((1, H), lambda i: (0, 0)),
            pl.BlockSpec((D, H), lambda i: (0, 0)),
            pl.BlockSpec((D, H), lambda i: (0, 0)),
            pl.BlockSpec((1, H), lambda i: (0, 0)),
        ],
        out_specs=[
            pl.BlockSpec((bpre, GW), lambda i: (i, 0)),
            pl.BlockSpec((bpre, GW), lambda i: (i, 0)),
            pl.BlockSpec((bpre, H), lambda i: (i, 0)),
        ],
        out_shape=[
            jax.ShapeDtypeStruct((N, GW), jnp.float32),
            jax.ShapeDtypeStruct((N, GW), jnp.float32),
            jax.ShapeDtypeStruct((N, H), jnp.float32),
        ],
    )(x, posp, w1a, b1.reshape(1, H), w1b, wn1a, bn1.reshape(1, H))

    ga, gb = _make_gather(E, N)(xap, xbp, row, col)

    bedge = 2000
    md = pl.pallas_call(
        _edge_body,
        grid=(E // bedge,),
        in_specs=[
            pl.BlockSpec((bedge, GW), lambda i: (i, 0)),
            pl.BlockSpec((bedge, GW), lambda i: (i, 0)),
            pl.BlockSpec((bedge, 16), lambda i: (i, 0)),
            pl.BlockSpec((1, H), lambda i: (0, 0)),
            pl.BlockSpec((16, H), lambda i: (0, 0)),
            pl.BlockSpec((H, H), lambda i: (0, 0)),
            pl.BlockSpec((1, H), lambda i: (0, 0)),
            pl.BlockSpec((H, H), lambda i: (0, 0)),
            pl.BlockSpec((1, H), lambda i: (0, 0)),
            pl.BlockSpec((H, 16), lambda i: (0, 0)),
            pl.BlockSpec((1, 16), lambda i: (0, 0)),
        ],
        out_specs=pl.BlockSpec((bedge, GW), lambda i: (i, 0)),
        out_shape=jax.ShapeDtypeStruct((E, GW), jnp.float32),
    )(ga, gb, edge_attr, w1c, w1d, W2, b2.reshape(1, H),
      Wc1, bc1.reshape(1, H), wc2t, bc2t)

    zeros = jnp.zeros((N, GW), jnp.float32)
    accs = _make_scatter(E, N)(md, row, zeros)

    bnode = 2000
    x_new, posn = pl.pallas_call(
        _node_body,
        grid=(N // bnode,),
        in_specs=[
            pl.BlockSpec((bnode, GW), lambda i: (i, 0)),
            pl.BlockSpec((bnode, GW), lambda i: (i, 0)),
            pl.BlockSpec((bnode, H), lambda i: (i, 0)),
            pl.BlockSpec((bnode, 16), lambda i: (i, 0)),
            pl.BlockSpec((H, H), lambda i: (0, 0)),
            pl.BlockSpec((H, D), lambda i: (0, 0)),
            pl.BlockSpec((1, D), lambda i: (0, 0)),
        ],
        out_specs=[
            pl.BlockSpec((bnode, D), lambda i: (i, 0)),
            pl.BlockSpec((bnode, 16), lambda i: (i, 0)),
        ],
        out_shape=[
            jax.ShapeDtypeStruct((N, D), jnp.float32),
            jax.ShapeDtypeStruct((N, 16), jnp.float32),
        ],
    )(accs[0], accs[1], xn, posp, wn1b, Wn2, bn2.reshape(1, D))

    return (x_new, posn[:, :3])


# speedup vs baseline: 4.1996x; 1.4654x over previous
import functools

import jax
import jax.numpy as jnp
from jax import lax
from jax.experimental import pallas as pl
from jax.experimental.pallas import tpu as pltpu
from jax.experimental.pallas import tpu_sc as plsc

NC = 2
NS = 16
NW = NC * NS
GW = 80
CHUNK = 125
NBUF = 4


def _silu(v):
    return v / (1.0 + jnp.exp(-v))


def _pre_body(x_ref, posp_ref, w1a_ref, b1_ref, w1b_ref, wn1a_ref, bn1_ref,
              xap_ref, xbp_ref, xn_ref):
    x = x_ref[...]
    posp = posp_ref[...]
    xa = jnp.dot(x, w1a_ref[...], preferred_element_type=jnp.float32) + b1_ref[...]
    xb = jnp.dot(x, w1b_ref[...], preferred_element_type=jnp.float32)
    xap_ref[...] = jnp.concatenate([xa, posp], axis=1)
    xbp_ref[...] = jnp.concatenate([xb, posp], axis=1)
    xn_ref[...] = jnp.dot(x, wn1a_ref[...], preferred_element_type=jnp.float32) + bn1_ref[...]


def _edge_body(ga_ref, gb_ref, ea_ref, ones_ref, w1c_ref, w1d_ref, w2_ref,
               b2_ref, wc1_ref, bc1_ref, wc2_ref, bc2_ref, md_ref):
    ga = ga_ref[...]
    gb = gb_ref[...]
    diffp = ga[:, 64:80] - gb[:, 64:80]
    sq = jnp.dot(diffp * diffp, ones_ref[...], preferred_element_type=jnp.float32)
    dist = jnp.sqrt(sq)
    pre = (ga[:, :64] + gb[:, :64]
           + dist * w1c_ref[...]
           + jnp.dot(ea_ref[...], w1d_ref[...], preferred_element_type=jnp.float32))
    msg = _silu(jnp.dot(_silu(pre), w2_ref[...], preferred_element_type=jnp.float32)
                + b2_ref[...])
    c1 = _silu(jnp.dot(msg, wc1_ref[...], preferred_element_type=jnp.float32)
               + bc1_ref[...])
    cw = jnp.dot(c1, wc2_ref[...], preferred_element_type=jnp.float32) + bc2_ref[...]
    md_ref[...] = jnp.concatenate([msg, diffp * cw], axis=1)


def _node_body(acc0_ref, acc1_ref, xn_ref, posp_ref, wn1b_ref, wn2_ref, bn2_ref,
               xnew_ref, posn_ref):
    acc = acc0_ref[...] + acc1_ref[...]
    h = _silu(xn_ref[...] + jnp.dot(acc[:, :64], wn1b_ref[...],
                                    preferred_element_type=jnp.float32))
    xnew_ref[...] = jnp.dot(h, wn2_ref[...], preferred_element_type=jnp.float32) + bn2_ref[...]
    posn_ref[...] = posp_ref[...] + acc[:, 64:80]


def _make_gather(E, N):
    epw = E // NW
    nch = epw // CHUNK
    assert nch % NBUF == 0
    mesh = plsc.VectorSubcoreMesh(core_axis_name="c", subcore_axis_name="s",
                                  num_cores=NC, num_subcores=NS)

    @functools.partial(
        pl.kernel, mesh=mesh,
        compiler_params=pltpu.CompilerParams(use_tc_tiling_on_sc=False),
        out_type=[jax.ShapeDtypeStruct((E, GW), jnp.float32),
                  jax.ShapeDtypeStruct((E, GW), jnp.float32)],
        scratch_types=[pltpu.VMEM((nch, CHUNK), jnp.int32),
                       pltpu.VMEM((nch, CHUNK), jnp.int32)]
                      + [pltpu.VMEM((CHUNK, GW), jnp.float32)] * (2 * NBUF)
                      + [pltpu.SemaphoreType.DMA] * (2 * NBUF),
    )
    def gather_k(xap_hbm, xbp_hbm, row2_hbm, col2_hbm, ga_hbm, gb_hbm,
                 idx_r, idx_c, *bufs_sems):
        buf_a = bufs_sems[0:NBUF]
        buf_b = bufs_sems[NBUF:2 * NBUF]
        sem_a = bufs_sems[2 * NBUF:3 * NBUF]
        sem_b = bufs_sems[3 * NBUF:4 * NBUF]
        wid = lax.axis_index("s") * NC + lax.axis_index("c")
        base = wid * epw
        pltpu.sync_copy(row2_hbm.at[wid], idx_r)
        pltpu.sync_copy(col2_hbm.at[wid], idx_c)
        for b in range(NBUF):
            pltpu.async_copy(xap_hbm.at[idx_r.at[b]], buf_a[b], sem_a[b])
            pltpu.async_copy(xbp_hbm.at[idx_c.at[b]], buf_b[b], sem_b[b])

        def body(j, carry):
            for b in range(NBUF):
                i = j * NBUF + b
                off = base + i * CHUNK
                pltpu.make_async_copy(xap_hbm.at[idx_r.at[0]], buf_a[b],
                                      sem_a[b]).wait()
                pltpu.make_async_copy(xbp_hbm.at[idx_c.at[0]], buf_b[b],
                                      sem_b[b]).wait()
                pltpu.sync_copy(buf_a[b], ga_hbm.at[pl.ds(off, CHUNK)])
                pltpu.sync_copy(buf_b[b], gb_hbm.at[pl.ds(off, CHUNK)])

                @pl.when(i + NBUF < nch)
                def _():
                    pltpu.async_copy(xap_hbm.at[idx_r.at[i + NBUF]],
                                     buf_a[b], sem_a[b])
                    pltpu.async_copy(xbp_hbm.at[idx_c.at[i + NBUF]],
                                     buf_b[b], sem_b[b])
            return carry

        lax.fori_loop(0, nch // NBUF, body, 0)

    return gather_k


def _make_scatter(E, N):
    epw = E // NW
    nch = epw // CHUNK
    assert nch % NBUF == 0
    npc = N // NS
    mesh = plsc.VectorSubcoreMesh(core_axis_name="c", subcore_axis_name="s",
                                  num_cores=NC, num_subcores=NS)

    @functools.partial(
        pl.kernel, mesh=mesh,
        compiler_params=pltpu.CompilerParams(use_tc_tiling_on_sc=False),
        out_type=jax.ShapeDtypeStruct((NC, N, GW), jnp.float32),
        scratch_types=[pltpu.VMEM((nch, CHUNK), jnp.int32),
                       pltpu.VMEM_SHARED((N, GW), jnp.float32)]
                      + [pltpu.VMEM((CHUNK, GW), jnp.float32)] * NBUF
                      + [pltpu.SemaphoreType.DMA] * NBUF,
    )
    def scatter_k(md_hbm, row2_hbm, zeros_hbm, acc_hbm, idx_v, acc_sh,
                  *bufs_sems):
        bufs = bufs_sems[0:NBUF]
        sems = bufs_sems[NBUF:2 * NBUF]
        cid = lax.axis_index("c")
        sid = lax.axis_index("s")
        wid = sid * NC + cid
        base = wid * epw
        pltpu.sync_copy(row2_hbm.at[wid], idx_v)
        pltpu.sync_copy(zeros_hbm.at[pl.ds(sid * npc, npc)],
                        acc_sh.at[pl.ds(sid * npc, npc)])
        for b in range(NBUF):
            pltpu.async_copy(md_hbm.at[pl.ds(base + b * CHUNK, CHUNK)],
                             bufs[b], sems[b])
        plsc.subcore_barrier()

        def body(j, carry):
            for b in range(NBUF):
                i = j * NBUF + b
                pltpu.make_async_copy(md_hbm.at[pl.ds(base, CHUNK)], bufs[b],
                                      sems[b]).wait()
                pltpu.sync_copy(bufs[b], acc_sh.at[idx_v.at[i]], add=True)

                @pl.when(i + NBUF < nch)
                def _():
                    pltpu.async_copy(
                        md_hbm.at[pl.ds(base + (i + NBUF) * CHUNK, CHUNK)],
                        bufs[b], sems[b])
            return carry

        lax.fori_loop(0, nch // NBUF, body, 0)
        plsc.subcore_barrier()
        pltpu.sync_copy(acc_sh.at[pl.ds(sid * npc, npc)],
                        acc_hbm.at[cid, pl.ds(sid * npc, npc)])

    return scatter_k


def kernel(x, pos, edge_index, edge_attr, W1, b1, W2, b2,
           Wn1, bn1, Wn2, bn2, Wc1, bc1, Wc2, bc2):
    N, D = x.shape
    E = edge_index.shape[1]
    H = W2.shape[0]
    assert D == 128 and H == 64
    assert E % (NW * CHUNK) == 0 and N % NS == 0

    epw = E // NW
    nch = epw // CHUNK
    row = edge_index[0]
    col = edge_index[1]
    row2 = row.reshape(NW, nch, CHUNK)
    col2 = edge_index[1].reshape(NW, nch, CHUNK)
    posp = jnp.pad(pos, ((0, 0), (0, 16 - pos.shape[1])))
    w1a = W1[:D]
    w1b = W1[D:2 * D]
    w1c = W1[2 * D:2 * D + 1]
    w1d = W1[2 * D + 1:]
    wn1a = Wn1[:D]
    wn1b = Wn1[D:]
    wc2t = jnp.tile(Wc2, (1, 16))
    bc2t = jnp.broadcast_to(bc2.reshape(1, 1), (1, 16))

    bpre = 2000
    xap, xbp, xn = pl.pallas_call(
        _pre_body,
        grid=(N // bpre,),
        in_specs=[
            pl.BlockSpec((bpre, D), lambda i: (i, 0)),
            pl.BlockSpec((bpre, 16), lambda i: (i, 0)),
            pl.BlockSpec((D, H), lambda i: (0, 0)),
            pl.BlockSpec((1, H), lambda i: (0, 0)),
            pl.BlockSpec((D, H), lambda i: (0, 0)),
            pl.BlockSpec((D, H), lambda i: (0, 0)),
            pl.BlockSpec((1, H), lambda i: (0, 0)),
        ],
        out_specs=[
            pl.BlockSpec((bpre, GW), lambda i: (i, 0)),
            pl.BlockSpec((bpre, GW), lambda i: (i, 0)),
            pl.BlockSpec((bpre, H), lambda i: (i, 0)),
        ],
        out_shape=[
            jax.ShapeDtypeStruct((N, GW), jnp.float32),
            jax.ShapeDtypeStruct((N, GW), jnp.float32),
            jax.ShapeDtypeStruct((N, H), jnp.float32),
        ],
    )(x, posp, w1a, b1.reshape(1, H), w1b, wn1a, bn1.reshape(1, H))

    ga, gb = _make_gather(E, N)(xap, xbp, row2, col2)

    bedge = 2000
    md = pl.pallas_call(
        _edge_body,
        grid=(E // bedge,),
        in_specs=[
            pl.BlockSpec((bedge, GW), lambda i: (i, 0)),
            pl.BlockSpec((bedge, GW), lambda i: (i, 0)),
            pl.BlockSpec((bedge, 16), lambda i: (i, 0)),
            pl.BlockSpec((16, H), lambda i: (0, 0)),
            pl.BlockSpec((1, H), lambda i: (0, 0)),
            pl.BlockSpec((16, H), lambda i: (0, 0)),
            pl.BlockSpec((H, H), lambda i: (0, 0)),
            pl.BlockSpec((1, H), lambda i: (0, 0)),
            pl.BlockSpec((H, H), lambda i: (0, 0)),
            pl.BlockSpec((1, H), lambda i: (0, 0)),
            pl.BlockSpec((H, 16), lambda i: (0, 0)),
            pl.BlockSpec((1, 16), lambda i: (0, 0)),
        ],
        out_specs=pl.BlockSpec((bedge, GW), lambda i: (i, 0)),
        out_shape=jax.ShapeDtypeStruct((E, GW), jnp.float32),
    )(ga, gb, edge_attr, jnp.ones((16, H), jnp.float32), w1c, w1d, W2,
      b2.reshape(1, H), Wc1, bc1.reshape(1, H), wc2t, bc2t)

    zeros = jnp.zeros((N, GW), jnp.float32)
    accs = _make_scatter(E, N)(md, row2, zeros)

    bnode = 2000
    x_new, posn = pl.pallas_call(
        _node_body,
        grid=(N // bnode,),
        in_specs=[
            pl.BlockSpec((bnode, GW), lambda i: (i, 0)),
            pl.BlockSpec((bnode, GW), lambda i: (i, 0)),
            pl.BlockSpec((bnode, H), lambda i: (i, 0)),
            pl.BlockSpec((bnode, 16), lambda i: (i, 0)),
            pl.BlockSpec((H, H), lambda i: (0, 0)),
            pl.BlockSpec((H, D), lambda i: (0, 0)),
            pl.BlockSpec((1, D), lambda i: (0, 0)),
        ],
        out_specs=[
            pl.BlockSpec((bnode, D), lambda i: (i, 0)),
            pl.BlockSpec((bnode, 16), lambda i: (i, 0)),
        ],
        out_shape=[
            jax.ShapeDtypeStruct((N, D), jnp.float32),
            jax.ShapeDtypeStruct((N, 16), jnp.float32),
        ],
    )(accs[0], accs[1], xn, posp, wn1b, Wn2, bn2.reshape(1, D))

    return (x_new, posn[:, :3])


# speedup vs baseline: 6.2998x; 1.5001x over previous
import functools

import jax
import jax.numpy as jnp
from jax import lax
from jax.experimental import pallas as pl
from jax.experimental.pallas import tpu as pltpu
from jax.experimental.pallas import tpu_sc as plsc

NC = 2
NS = 16
NW = NC * NS
GW = 128
CHUNK = 80
NBUF = 4


def _silu(v):
    return v / (1.0 + jnp.exp(-v))


def _pre_body(x_ref, posp_ref, w1a_ref, b1_ref, w1b_ref, wn1a_ref, bn1_ref,
              xap_ref, xbp_ref, xn_ref):
    x = x_ref[...]
    posp = posp_ref[...]
    z = jnp.zeros((x.shape[0], GW - 80), jnp.float32)
    xa = jnp.dot(x, w1a_ref[...], preferred_element_type=jnp.float32) + b1_ref[...]
    xb = jnp.dot(x, w1b_ref[...], preferred_element_type=jnp.float32)
    xap_ref[...] = jnp.concatenate([xa, posp, z], axis=1)
    xbp_ref[...] = jnp.concatenate([xb, posp, z], axis=1)
    xn_ref[...] = jnp.dot(x, wn1a_ref[...], preferred_element_type=jnp.float32) + bn1_ref[...]


def _edge_body(ga_ref, gb_ref, ea_ref, ones_ref, w1c_ref, w1d_ref, w2_ref,
               b2_ref, wc1_ref, bc1_ref, wc2_ref, bc2_ref, md_ref):
    ga = ga_ref[...]
    gb = gb_ref[...]
    diffp = ga[:, 64:80] - gb[:, 64:80]
    sq = jnp.dot(diffp * diffp, ones_ref[...], preferred_element_type=jnp.float32)
    dist = jnp.sqrt(sq)
    pre = (ga[:, :64] + gb[:, :64]
           + dist * w1c_ref[...]
           + jnp.dot(ea_ref[...], w1d_ref[...], preferred_element_type=jnp.float32))
    msg = _silu(jnp.dot(_silu(pre), w2_ref[...], preferred_element_type=jnp.float32)
                + b2_ref[...])
    c1 = _silu(jnp.dot(msg, wc1_ref[...], preferred_element_type=jnp.float32)
               + bc1_ref[...])
    cw = jnp.dot(c1, wc2_ref[...], preferred_element_type=jnp.float32) + bc2_ref[...]
    z = jnp.zeros((ga.shape[0], GW - 80), jnp.float32)
    md_ref[...] = jnp.concatenate([msg, diffp * cw, z], axis=1)


def _node_body(acc0_ref, acc1_ref, xn_ref, posp_ref, wn1b_ref, wn2_ref, bn2_ref,
               xnew_ref, posn_ref):
    acc = acc0_ref[...] + acc1_ref[...]
    h = _silu(xn_ref[...] + jnp.dot(acc[:, :64], wn1b_ref[...],
                                    preferred_element_type=jnp.float32))
    xnew_ref[...] = jnp.dot(h, wn2_ref[...], preferred_element_type=jnp.float32) + bn2_ref[...]
    posn_ref[...] = posp_ref[...] + acc[:, 64:80]


def _make_gather(E, N):
    epw = E // NW
    nch = epw // CHUNK
    nmain = (nch - 1) // NBUF * NBUF
    mesh = plsc.VectorSubcoreMesh(core_axis_name="c", subcore_axis_name="s",
                                  num_cores=NC, num_subcores=NS)

    @functools.partial(
        pl.kernel, mesh=mesh,
        out_type=[jax.ShapeDtypeStruct((E, GW), jnp.float32),
                  jax.ShapeDtypeStruct((E, GW), jnp.float32)],
        scratch_types=[pltpu.VMEM((epw,), jnp.int32),
                       pltpu.VMEM((epw,), jnp.int32)]
                      + [pltpu.VMEM((CHUNK, GW), jnp.float32)] * (2 * NBUF)
                      + [pltpu.SemaphoreType.DMA] * (2 * NBUF),
    )
    def gather_k(xap_hbm, xbp_hbm, row_hbm, col_hbm, ga_hbm, gb_hbm,
                 idx_r, idx_c, *bufs_sems):
        buf_a = bufs_sems[0:NBUF]
        buf_b = bufs_sems[NBUF:2 * NBUF]
        sem_a = bufs_sems[2 * NBUF:3 * NBUF]
        sem_b = bufs_sems[3 * NBUF:4 * NBUF]
        wid = lax.axis_index("s") * NC + lax.axis_index("c")
        base = wid * epw
        pltpu.sync_copy(row_hbm.at[pl.ds(base, epw)], idx_r)
        pltpu.sync_copy(col_hbm.at[pl.ds(base, epw)], idx_c)

        def fire(i, b):
            pltpu.async_copy(xap_hbm.at[idx_r.at[pl.ds(i * CHUNK, CHUNK)]],
                             buf_a[b], sem_a[b])
            pltpu.async_copy(xbp_hbm.at[idx_c.at[pl.ds(i * CHUNK, CHUNK)]],
                             buf_b[b], sem_b[b])

        def drain_store(i, b):
            off = base + i * CHUNK
            pltpu.make_async_copy(xap_hbm.at[idx_r.at[pl.ds(0, CHUNK)]],
                                  buf_a[b], sem_a[b]).wait()
            pltpu.make_async_copy(xbp_hbm.at[idx_c.at[pl.ds(0, CHUNK)]],
                                  buf_b[b], sem_b[b]).wait()
            pltpu.sync_copy(buf_a[b], ga_hbm.at[pl.ds(off, CHUNK)])
            pltpu.sync_copy(buf_b[b], gb_hbm.at[pl.ds(off, CHUNK)])

        for b in range(NBUF):
            fire(b, b)

        def body(j, carry):
            for b in range(NBUF):
                i = j * NBUF + b
                drain_store(i, b)

                @pl.when(i + NBUF < nch)
                def _():
                    fire(i + NBUF, b)
            return carry

        lax.fori_loop(0, nmain // NBUF, body, 0)
        for i in range(nmain, nch):
            drain_store(i, i % NBUF)

    return gather_k


def _make_scatter(E, N):
    epw = E // NW
    nch = epw // CHUNK
    nmain = (nch - 1) // NBUF * NBUF
    npc = -(-N // NS) // 8 * 8
    npc_last = N - npc * (NS - 1)
    assert npc_last > 0
    mesh = plsc.VectorSubcoreMesh(core_axis_name="c", subcore_axis_name="s",
                                  num_cores=NC, num_subcores=NS)

    @functools.partial(
        pl.kernel, mesh=mesh,
        out_type=jax.ShapeDtypeStruct((NC, N, GW), jnp.float32),
        scratch_types=[pltpu.VMEM_SHARED((N, GW), jnp.float32)]
                      + [pltpu.VMEM((CHUNK,), jnp.int32)] * NBUF
                      + [pltpu.VMEM((CHUNK, GW), jnp.float32)] * NBUF
                      + [pltpu.SemaphoreType.DMA] * (2 * NBUF),
    )
    def scatter_k(md_hbm, row_hbm, zeros_hbm, acc_hbm, acc_sh, *bufs_sems):
        idxs = bufs_sems[0:NBUF]
        bufs = bufs_sems[NBUF:2 * NBUF]
        isems = bufs_sems[2 * NBUF:3 * NBUF]
        dsems = bufs_sems[3 * NBUF:4 * NBUF]
        cid = lax.axis_index("c")
        sid = lax.axis_index("s")
        wid = sid * NC + cid
        base = wid * epw

        def fire(i, b):
            pltpu.async_copy(row_hbm.at[pl.ds(base + i * CHUNK, CHUNK)],
                             idxs[b], isems[b])
            pltpu.async_copy(md_hbm.at[pl.ds(base + i * CHUNK, CHUNK)],
                             bufs[b], dsems[b])

        def drain_scatter(i, b):
            pltpu.make_async_copy(row_hbm.at[pl.ds(0, CHUNK)], idxs[b],
                                  isems[b]).wait()
            pltpu.make_async_copy(md_hbm.at[pl.ds(0, CHUNK)], bufs[b],
                                  dsems[b]).wait()
            pltpu.sync_copy(bufs[b], acc_sh.at[idxs[b]], add=True)

        @pl.when(sid < NS - 1)
        def _():
            pltpu.sync_copy(zeros_hbm.at[pl.ds(sid * npc, npc)],
                            acc_sh.at[pl.ds(sid * npc, npc)])

        @pl.when(sid == NS - 1)
        def _():
            pltpu.sync_copy(zeros_hbm.at[pl.ds((NS - 1) * npc, npc_last)],
                            acc_sh.at[pl.ds((NS - 1) * npc, npc_last)])

        for b in range(NBUF):
            fire(b, b)
        plsc.subcore_barrier()

        def body(j, carry):
            for b in range(NBUF):
                i = j * NBUF + b
                drain_scatter(i, b)

                @pl.when(i + NBUF < nch)
                def _():
                    fire(i + NBUF, b)
            return carry

        lax.fori_loop(0, nmain // NBUF, body, 0)
        for i in range(nmain, nch):
            drain_scatter(i, i % NBUF)
        plsc.subcore_barrier()

        @pl.when(sid < NS - 1)
        def _():
            pltpu.sync_copy(acc_sh.at[pl.ds(sid * npc, npc)],
                            acc_hbm.at[cid, pl.ds(sid * npc, npc)])

        @pl.when(sid == NS - 1)
        def _():
            pltpu.sync_copy(acc_sh.at[pl.ds((NS - 1) * npc, npc_last)],
                            acc_hbm.at[cid, pl.ds((NS - 1) * npc, npc_last)])

    return scatter_k


def kernel(x, pos, edge_index, edge_attr, W1, b1, W2, b2,
           Wn1, bn1, Wn2, bn2, Wc1, bc1, Wc2, bc2):
    N, D = x.shape
    E = edge_index.shape[1]
    H = W2.shape[0]
    assert D == 128 and H == 64
    assert E % (NW * CHUNK) == 0 and N % NS == 0

    row = edge_index[0]
    col = edge_index[1]
    posp = jnp.pad(pos, ((0, 0), (0, 16 - pos.shape[1])))
    w1a = W1[:D]
    w1b = W1[D:2 * D]
    w1c = W1[2 * D:2 * D + 1]
    w1d = W1[2 * D + 1:]
    wn1a = Wn1[:D]
    wn1b = Wn1[D:]
    wc2t = jnp.tile(Wc2, (1, 16))
    bc2t = jnp.broadcast_to(bc2.reshape(1, 1), (1, 16))

    bpre = 2000
    xap, xbp, xn = pl.pallas_call(
        _pre_body,
        grid=(N // bpre,),
        in_specs=[
            pl.BlockSpec((bpre, D), lambda i: (i, 0)),
            pl.BlockSpec((bpre, 16), lambda i: (i, 0)),
            pl.BlockSpec((D, H), lambda i: (0, 0)),
            pl.BlockSpec((1, H), lambda i: (0, 0)),
            pl.BlockSpec((D, H), lambda i: (0, 0)),
            pl.BlockSpec((D, H), lambda i: (0, 0)),
            pl.BlockSpec((1, H), lambda i: (0, 0)),
        ],
        out_specs=[
            pl.BlockSpec((bpre, GW), lambda i: (i, 0)),
            pl.BlockSpec((bpre, GW), lambda i: (i, 0)),
            pl.BlockSpec((bpre, H), lambda i: (i, 0)),
        ],
        out_shape=[
            jax.ShapeDtypeStruct((N, GW), jnp.float32),
            jax.ShapeDtypeStruct((N, GW), jnp.float32),
            jax.ShapeDtypeStruct((N, H), jnp.float32),
        ],
    )(x, posp, w1a, b1.reshape(1, H), w1b, wn1a, bn1.reshape(1, H))

    ga, gb = _make_gather(E, N)(xap, xbp, row, col)

    bedge = 2000
    md = pl.pallas_call(
        _edge_body,
        grid=(E // bedge,),
        in_specs=[
            pl.BlockSpec((bedge, GW), lambda i: (i, 0)),
            pl.BlockSpec((bedge, GW), lambda i: (i, 0)),
            pl.BlockSpec((bedge, 16), lambda i: (i, 0)),
            pl.BlockSpec((16, H), lambda i: (0, 0)),
            pl.BlockSpec((1, H), lambda i: (0, 0)),
            pl.BlockSpec((16, H), lambda i: (0, 0)),
            pl.BlockSpec((H, H), lambda i: (0, 0)),
            pl.BlockSpec((1, H), lambda i: (0, 0)),
            pl.BlockSpec((H, H), lambda i: (0, 0)),
            pl.BlockSpec((1, H), lambda i: (0, 0)),
            pl.BlockSpec((H, 16), lambda i: (0, 0)),
            pl.BlockSpec((1, 16), lambda i: (0, 0)),
        ],
        out_specs=pl.BlockSpec((bedge, GW), lambda i: (i, 0)),
        out_shape=jax.ShapeDtypeStruct((E, GW), jnp.float32),
    )(ga, gb, edge_attr, jnp.ones((16, H), jnp.float32), w1c, w1d, W2,
      b2.reshape(1, H), Wc1, bc1.reshape(1, H), wc2t, bc2t)

    zeros = jnp.zeros((N, GW), jnp.float32)
    accs = _make_scatter(E, N)(md, row, zeros)

    bnode = 2000
    x_new, posn = pl.pallas_call(
        _node_body,
        grid=(N // bnode,),
        in_specs=[
            pl.BlockSpec((bnode, GW), lambda i: (i, 0)),
            pl.BlockSpec((bnode, GW), lambda i: (i, 0)),
            pl.BlockSpec((bnode, H), lambda i: (i, 0)),
            pl.BlockSpec((bnode, 16), lambda i: (i, 0)),
            pl.BlockSpec((H, H), lambda i: (0, 0)),
            pl.BlockSpec((H, D), lambda i: (0, 0)),
            pl.BlockSpec((1, D), lambda i: (0, 0)),
        ],
        out_specs=[
            pl.BlockSpec((bnode, D), lambda i: (i, 0)),
            pl.BlockSpec((bnode, 16), lambda i: (i, 0)),
        ],
        out_shape=[
            jax.ShapeDtypeStruct((N, D), jnp.float32),
            jax.ShapeDtypeStruct((N, 16), jnp.float32),
        ],
    )(accs[0], accs[1], xn, posp, wn1b, Wn2, bn2.reshape(1, D))

    return (x_new, posn[:, :3])


# speedup vs baseline: 8.2655x; 1.3120x over previous
import functools

import jax
import jax.numpy as jnp
from jax import lax
from jax.experimental import pallas as pl
from jax.experimental.pallas import tpu as pltpu
from jax.experimental.pallas import tpu_sc as plsc

NC = 2
NS = 16
NW = NC * NS
GW = 128
CHUNK = 40
NBUF = 4


def _silu(v):
    return v / (1.0 + jnp.exp(-v))


def _pre_body(x_ref, posp_ref, w1a_ref, b1_ref, w1b_ref, wn1a_ref, bn1_ref,
              xap_ref, xbp_ref, xn_ref):
    x = x_ref[...]
    posp = posp_ref[...]
    z = jnp.zeros((x.shape[0], GW - 80), jnp.float32)
    xa = jnp.dot(x, w1a_ref[...], preferred_element_type=jnp.float32) + b1_ref[...]
    xb = jnp.dot(x, w1b_ref[...], preferred_element_type=jnp.float32)
    xap_ref[...] = jnp.concatenate([xa, posp, z], axis=1)
    xbp_ref[...] = jnp.concatenate([xb, posp, z], axis=1)
    xn_ref[...] = jnp.dot(x, wn1a_ref[...], preferred_element_type=jnp.float32) + bn1_ref[...]


def _edge_body(ga_ref, gb_ref, eat_ref, ones_ref, w1c_ref, w1d_ref, w2_ref,
               b2_ref, wc1_ref, bc1_ref, wc2_ref, bc2_ref, md_ref):
    ga = ga_ref[...]
    gb = gb_ref[...]
    diffp = ga[:, 64:80] - gb[:, 64:80]
    sq = jnp.dot(diffp * diffp, ones_ref[...], preferred_element_type=jnp.float32)
    dist = jnp.sqrt(sq)
    eterm = lax.dot_general(eat_ref[...], w1d_ref[...],
                            (((0,), (0,)), ((), ())),
                            preferred_element_type=jnp.float32)
    pre = ga[:, :64] + gb[:, :64] + dist * w1c_ref[...] + eterm
    msg = _silu(jnp.dot(_silu(pre), w2_ref[...], preferred_element_type=jnp.float32)
                + b2_ref[...])
    c1 = _silu(jnp.dot(msg, wc1_ref[...], preferred_element_type=jnp.float32)
               + bc1_ref[...])
    cw = jnp.dot(c1, wc2_ref[...], preferred_element_type=jnp.float32) + bc2_ref[...]
    z = jnp.zeros((ga.shape[0], GW - 80), jnp.float32)
    md_ref[...] = jnp.concatenate([msg, diffp * cw, z], axis=1)


def _node_body(acc0_ref, acc1_ref, acc2_ref, acc3_ref, xn_ref, posp_ref,
               wn1b_ref, wn2_ref, bn2_ref, xnew_ref, posn_ref):
    acc = ((acc0_ref[...] + acc1_ref[...])
           + (acc2_ref[...] + acc3_ref[...]))
    h = _silu(xn_ref[...] + jnp.dot(acc[:, :64], wn1b_ref[...],
                                    preferred_element_type=jnp.float32))
    xnew_ref[...] = jnp.dot(h, wn2_ref[...], preferred_element_type=jnp.float32) + bn2_ref[...]
    posn_ref[...] = posp_ref[...] + acc[:, 64:80]


def _make_gather(E, N):
    epw = E // NW
    nch = epw // CHUNK
    nmain = (nch - 1) // NBUF * NBUF
    mesh = plsc.VectorSubcoreMesh(core_axis_name="c", subcore_axis_name="s",
                                  num_cores=NC, num_subcores=NS)

    @functools.partial(
        pl.kernel, mesh=mesh,
        out_type=[jax.ShapeDtypeStruct((E, GW), jnp.float32),
                  jax.ShapeDtypeStruct((E, GW), jnp.float32)],
        scratch_types=[pltpu.VMEM((epw,), jnp.int32),
                       pltpu.VMEM((epw,), jnp.int32)]
                      + [pltpu.VMEM((CHUNK, GW), jnp.float32)] * (2 * NBUF)
                      + [pltpu.SemaphoreType.DMA] * (2 * NBUF),
    )
    def gather_k(xap_hbm, xbp_hbm, row_hbm, col_hbm, ga_hbm, gb_hbm,
                 idx_r, idx_c, *bufs_sems):
        buf_a = bufs_sems[0:NBUF]
        buf_b = bufs_sems[NBUF:2 * NBUF]
        sem_a = bufs_sems[2 * NBUF:3 * NBUF]
        sem_b = bufs_sems[3 * NBUF:4 * NBUF]
        wid = lax.axis_index("s") * NC + lax.axis_index("c")
        base = wid * epw
        pltpu.sync_copy(row_hbm.at[pl.ds(base, epw)], idx_r)
        pltpu.sync_copy(col_hbm.at[pl.ds(base, epw)], idx_c)

        def fire(i, b):
            pltpu.async_copy(xap_hbm.at[idx_r.at[pl.ds(i * CHUNK, CHUNK)]],
                             buf_a[b], sem_a[b])
            pltpu.async_copy(xbp_hbm.at[idx_c.at[pl.ds(i * CHUNK, CHUNK)]],
                             buf_b[b], sem_b[b])

        def drain_store(i, b):
            off = base + i * CHUNK
            pltpu.make_async_copy(xap_hbm.at[idx_r.at[pl.ds(0, CHUNK)]],
                                  buf_a[b], sem_a[b]).wait()
            pltpu.make_async_copy(xbp_hbm.at[idx_c.at[pl.ds(0, CHUNK)]],
                                  buf_b[b], sem_b[b]).wait()
            pltpu.sync_copy(buf_a[b], ga_hbm.at[pl.ds(off, CHUNK)])
            pltpu.sync_copy(buf_b[b], gb_hbm.at[pl.ds(off, CHUNK)])

        for b in range(NBUF):
            fire(b, b)

        def body(j, carry):
            for b in range(NBUF):
                i = j * NBUF + b
                drain_store(i, b)

                @pl.when(i + NBUF < nch)
                def _():
                    fire(i + NBUF, b)
            return carry

        lax.fori_loop(0, nmain // NBUF, body, 0)
        for i in range(nmain, nch):
            drain_store(i, i % NBUF)

    return gather_k


def _make_scatter(E, N):
    epw = E // NW
    nch = epw // CHUNK
    nmain = (nch - 1) // NBUF * NBUF
    npc = -(-N // NS) // 8 * 8
    npc_last = N - npc * (NS - 1)
    assert npc_last > 0
    mesh = plsc.VectorSubcoreMesh(core_axis_name="c", subcore_axis_name="s",
                                  num_cores=NC, num_subcores=NS)

    @functools.partial(
        pl.kernel, mesh=mesh,
        out_type=jax.ShapeDtypeStruct((NC, N, GW), jnp.float32),
        scratch_types=[pltpu.VMEM_SHARED((N, GW), jnp.float32)]
                      + [pltpu.VMEM((CHUNK,), jnp.int32)] * NBUF
                      + [pltpu.VMEM((CHUNK, GW), jnp.float32)] * NBUF
                      + [pltpu.SemaphoreType.DMA] * (2 * NBUF),
    )
    def scatter_k(md_hbm, row_hbm, zeros_hbm, acc_hbm, acc_sh, *bufs_sems):
        idxs = bufs_sems[0:NBUF]
        bufs = bufs_sems[NBUF:2 * NBUF]
        isems = bufs_sems[2 * NBUF:3 * NBUF]
        dsems = bufs_sems[3 * NBUF:4 * NBUF]
        cid = lax.axis_index("c")
        sid = lax.axis_index("s")
        wid = sid * NC + cid
        base = wid * epw

        def fire(i, b):
            pltpu.async_copy(row_hbm.at[pl.ds(base + i * CHUNK, CHUNK)],
                             idxs[b], isems[b])
            pltpu.async_copy(md_hbm.at[pl.ds(base + i * CHUNK, CHUNK)],
                             bufs[b], dsems[b])

        def drain_scatter(i, b):
            pltpu.make_async_copy(row_hbm.at[pl.ds(0, CHUNK)], idxs[b],
                                  isems[b]).wait()
            pltpu.make_async_copy(md_hbm.at[pl.ds(0, CHUNK)], bufs[b],
                                  dsems[b]).wait()
            pltpu.sync_copy(bufs[b], acc_sh.at[idxs[b]], add=True)

        @pl.when(sid < NS - 1)
        def _():
            pltpu.sync_copy(zeros_hbm.at[pl.ds(sid * npc, npc)],
                            acc_sh.at[pl.ds(sid * npc, npc)])

        @pl.when(sid == NS - 1)
        def _():
            pltpu.sync_copy(zeros_hbm.at[pl.ds((NS - 1) * npc, npc_last)],
                            acc_sh.at[pl.ds((NS - 1) * npc, npc_last)])

        for b in range(NBUF):
            fire(b, b)
        plsc.subcore_barrier()

        def body(j, carry):
            for b in range(NBUF):
                i = j * NBUF + b
                drain_scatter(i, b)

                @pl.when(i + NBUF < nch)
                def _():
                    fire(i + NBUF, b)
            return carry

        lax.fori_loop(0, nmain // NBUF, body, 0)
        for i in range(nmain, nch):
            drain_scatter(i, i % NBUF)
        plsc.subcore_barrier()

        @pl.when(sid < NS - 1)
        def _():
            pltpu.sync_copy(acc_sh.at[pl.ds(sid * npc, npc)],
                            acc_hbm.at[cid, pl.ds(sid * npc, npc)])

        @pl.when(sid == NS - 1)
        def _():
            pltpu.sync_copy(acc_sh.at[pl.ds((NS - 1) * npc, npc_last)],
                            acc_hbm.at[cid, pl.ds((NS - 1) * npc, npc_last)])

    return scatter_k


def kernel(x, pos, edge_index, edge_attr, W1, b1, W2, b2,
           Wn1, bn1, Wn2, bn2, Wc1, bc1, Wc2, bc2):
    N, D = x.shape
    E = edge_index.shape[1]
    H = W2.shape[0]
    assert D == 128 and H == 64
    assert (E // 2) % (NW * CHUNK) == 0 and N % NS == 0

    row = edge_index[0]
    col = edge_index[1]
    posp = jnp.pad(pos, ((0, 0), (0, 16 - pos.shape[1])))
    w1a = W1[:D]
    w1b = W1[D:2 * D]
    w1c = W1[2 * D:2 * D + 1]
    w1d = W1[2 * D + 1:]
    wn1a = Wn1[:D]
    wn1b = Wn1[D:]
    wc2t = jnp.tile(Wc2, (1, 16))
    bc2t = jnp.broadcast_to(bc2.reshape(1, 1), (1, 16))

    bpre = 2000
    xap, xbp, xn = pl.pallas_call(
        _pre_body,
        grid=(N // bpre,),
        in_specs=[
            pl.BlockSpec((bpre, D), lambda i: (i, 0)),
            pl.BlockSpec((bpre, 16), lambda i: (i, 0)),
            pl.BlockSpec((D, H), lambda i: (0, 0)),
            pl.BlockSpec((1, H), lambda i: (0, 0)),
            pl.BlockSpec((D, H), lambda i: (0, 0)),
            pl.BlockSpec((D, H), lambda i: (0, 0)),
            pl.BlockSpec((1, H), lambda i: (0, 0)),
        ],
        out_specs=[
            pl.BlockSpec((bpre, GW), lambda i: (i, 0)),
            pl.BlockSpec((bpre, GW), lambda i: (i, 0)),
            pl.BlockSpec((bpre, H), lambda i: (i, 0)),
        ],
        out_shape=[
            jax.ShapeDtypeStruct((N, GW), jnp.float32),
            jax.ShapeDtypeStruct((N, GW), jnp.float32),
            jax.ShapeDtypeStruct((N, H), jnp.float32),
        ],
    )(x, posp, w1a, b1.reshape(1, H), w1b, wn1a, bn1.reshape(1, H))

    EH = E // 2
    eat = edge_attr.T
    zeros = jnp.zeros((N, GW), jnp.float32)
    bedge = 3200
    gather_fn = _make_gather(EH, N)
    scatter_fn = _make_scatter(EH, N)
    accs = []
    for s in range(2):
        row_h = lax.slice_in_dim(row, s * EH, (s + 1) * EH)
        col_h = lax.slice_in_dim(col, s * EH, (s + 1) * EH)
        ga, gb = gather_fn(xap, xbp, row_h, col_h)
        md = pl.pallas_call(
            _edge_body,
            grid=(EH // bedge,),
            in_specs=[
                pl.BlockSpec((bedge, GW), lambda i: (i, 0)),
                pl.BlockSpec((bedge, GW), lambda i: (i, 0)),
                pl.BlockSpec((16, bedge), lambda i, s=s: (0, i + s * (EH // bedge))),
                pl.BlockSpec((16, H), lambda i: (0, 0)),
                pl.BlockSpec((1, H), lambda i: (0, 0)),
                pl.BlockSpec((16, H), lambda i: (0, 0)),
                pl.BlockSpec((H, H), lambda i: (0, 0)),
                pl.BlockSpec((1, H), lambda i: (0, 0)),
                pl.BlockSpec((H, H), lambda i: (0, 0)),
                pl.BlockSpec((1, H), lambda i: (0, 0)),
                pl.BlockSpec((H, 16), lambda i: (0, 0)),
                pl.BlockSpec((1, 16), lambda i: (0, 0)),
            ],
            out_specs=pl.BlockSpec((bedge, GW), lambda i: (i, 0)),
            out_shape=jax.ShapeDtypeStruct((EH, GW), jnp.float32),
        )(ga, gb, eat, jnp.ones((16, H), jnp.float32), w1c, w1d, W2,
          b2.reshape(1, H), Wc1, bc1.reshape(1, H), wc2t, bc2t)
        acc_pair = scatter_fn(md, row_h, zeros)
        accs.extend([acc_pair[0], acc_pair[1]])

    bnode = 2000
    x_new, posn = pl.pallas_call(
        _node_body,
        grid=(N // bnode,),
        in_specs=[
            pl.BlockSpec((bnode, GW), lambda i: (i, 0)),
            pl.BlockSpec((bnode, GW), lambda i: (i, 0)),
            pl.BlockSpec((bnode, GW), lambda i: (i, 0)),
            pl.BlockSpec((bnode, GW), lambda i: (i, 0)),
            pl.BlockSpec((bnode, H), lambda i: (i, 0)),
            pl.BlockSpec((bnode, 16), lambda i: (i, 0)),
            pl.BlockSpec((H, H), lambda i: (0, 0)),
            pl.BlockSpec((H, D), lambda i: (0, 0)),
            pl.BlockSpec((1, D), lambda i: (0, 0)),
        ],
        out_specs=[
            pl.BlockSpec((bnode, D), lambda i: (i, 0)),
            pl.BlockSpec((bnode, 16), lambda i: (i, 0)),
        ],
        out_shape=[
            jax.ShapeDtypeStruct((N, D), jnp.float32),
            jax.ShapeDtypeStruct((N, 16), jnp.float32),
        ],
    )(accs[0], accs[1], accs[2], accs[3], xn, posp, wn1b, Wn2,
      bn2.reshape(1, D))

    return (x_new, posn[:, :3])
